# baseline (speedup 1.0000x reference)
import jax
import jax.numpy as jnp
from jax.experimental import pallas as pl

_N = 10000
_D = 128
_BM = 400


def _linear_kernel(x_ref, w_ref, b_ref, o_ref):
    o_ref[...] = (
        jnp.dot(x_ref[...], w_ref[...], preferred_element_type=jnp.float32)
        + b_ref[...]
    )


def _pass1_kernel(a_ref, h_ref, w2_ref, b2_ref, o_ref):
    acc = jnp.dot(a_ref[...], h_ref[...], preferred_element_type=jnp.float32)
    acc = jnp.maximum(acc, 0.0)
    o_ref[...] = (
        jnp.dot(acc, w2_ref[...], preferred_element_type=jnp.float32)
        + b2_ref[...]
    )


def _pass2_kernel(a_ref, h_ref, o_ref):
    o_ref[...] = jnp.dot(a_ref[...], h_ref[...], preferred_element_type=jnp.float32)


def kernel(x, adj_t, W1, b1, W2, b2):
    w1t = W1.T
    w2t = W2.T
    b1r = b1.reshape(1, _D)
    b2r = b2.reshape(1, _D)

    h1 = pl.pallas_call(
        _linear_kernel,
        out_shape=jax.ShapeDtypeStruct((_N, _D), jnp.float32),
        in_specs=[
            pl.BlockSpec((_N, _D), lambda: (0, 0)),
            pl.BlockSpec((_D, _D), lambda: (0, 0)),
            pl.BlockSpec((1, _D), lambda: (0, 0)),
        ],
        out_specs=pl.BlockSpec((_N, _D), lambda: (0, 0)),
    )(x, w1t, b1r)

    grid = (_N // _BM,)
    a_spec = pl.BlockSpec((_BM, _N), lambda i: (i, 0))
    h_spec = pl.BlockSpec((_N, _D), lambda i: (0, 0))
    o_spec = pl.BlockSpec((_BM, _D), lambda i: (i, 0))

    h2 = pl.pallas_call(
        _pass1_kernel,
        grid=grid,
        out_shape=jax.ShapeDtypeStruct((_N, _D), jnp.float32),
        in_specs=[
            a_spec,
            h_spec,
            pl.BlockSpec((_D, _D), lambda i: (0, 0)),
            pl.BlockSpec((1, _D), lambda i: (0, 0)),
        ],
        out_specs=o_spec,
    )(adj_t, h1, w2t, b2r)

    out = pl.pallas_call(
        _pass2_kernel,
        grid=grid,
        out_shape=jax.ShapeDtypeStruct((_N, _D), jnp.float32),
        in_specs=[a_spec, h_spec],
        out_specs=o_spec,
    )(adj_t, h2)

    return out


# speedup vs baseline: 1.0473x; 1.0473x over previous
import jax
import jax.numpy as jnp
from jax.experimental import pallas as pl
from jax.experimental.pallas import tpu as pltpu

_N = 10000
_D = 128
_BM = 400


def _gcn_kernel(x_ref, a_ref, w1_ref, b1_ref, w2_ref, b2_ref, o_ref,
                h1_ref, h2_ref):
    p = pl.program_id(0)
    i = pl.program_id(1)

    @pl.when(jnp.logical_and(p == 0, i == 0))
    def _():
        h1_ref[...] = (
            jnp.dot(x_ref[...], w1_ref[...], preferred_element_type=jnp.float32)
            + b1_ref[...]
        )

    @pl.when(p == 0)
    def _():
        acc = jnp.dot(a_ref[...], h1_ref[...], preferred_element_type=jnp.float32)
        h2 = (
            jnp.dot(jnp.maximum(acc, 0.0), w2_ref[...],
                    preferred_element_type=jnp.float32)
            + b2_ref[...]
        )
        h2_ref[pl.ds(i * _BM, _BM), :] = h2
        o_ref[...] = h2

    @pl.when(p == 1)
    def _():
        o_ref[...] = jnp.dot(a_ref[...], h2_ref[...],
                             preferred_element_type=jnp.float32)


def kernel(x, adj_t, W1, b1, W2, b2):
    w1t = W1.T
    w2t = W2.T
    b1r = b1.reshape(1, _D)
    b2r = b2.reshape(1, _D)

    return pl.pallas_call(
        _gcn_kernel,
        grid=(2, _N // _BM),
        out_shape=jax.ShapeDtypeStruct((_N, _D), jnp.float32),
        in_specs=[
            pl.BlockSpec((_N, _D), lambda p, i: (0, 0)),
            pl.BlockSpec((_BM, _N), lambda p, i: (i, 0)),
            pl.BlockSpec((_D, _D), lambda p, i: (0, 0)),
            pl.BlockSpec((1, _D), lambda p, i: (0, 0)),
            pl.BlockSpec((_D, _D), lambda p, i: (0, 0)),
            pl.BlockSpec((1, _D), lambda p, i: (0, 0)),
        ],
        out_specs=pl.BlockSpec((_BM, _D), lambda p, i: (i, 0)),
        scratch_shapes=[
            pltpu.VMEM((_N, _D), jnp.float32),
            pltpu.VMEM((_N, _D), jnp.float32),
        ],
        compiler_params=pltpu.CompilerParams(
            dimension_semantics=("arbitrary", "arbitrary"),
            vmem_limit_bytes=100 * 1024 * 1024,
        ),
    )(x, adj_t, w1t, b1r, w2t, b2r)


# speedup vs baseline: 1.0500x; 1.0026x over previous
import jax
import jax.numpy as jnp
from jax.experimental import pallas as pl
from jax.experimental.pallas import tpu as pltpu

_N = 10000
_D = 128
_BM = 400


def _gcn_kernel(x_ref, a_ref, w1_ref, b1_ref, w2_ref, b2_ref, o_ref,
                h1_ref, h2_ref):
    p = pl.program_id(0)
    i = pl.program_id(1)

    @pl.when(jnp.logical_and(p == 0, i == 0))
    def _():
        h1_ref[...] = (
            jnp.dot(x_ref[...], w1_ref[...], preferred_element_type=jnp.float32)
            + b1_ref[...]
        )

    @pl.when(p == 0)
    def _():
        acc = jnp.dot(a_ref[...], h1_ref[...], preferred_element_type=jnp.float32)
        h2 = (
            jnp.dot(jnp.maximum(acc, 0.0), w2_ref[...],
                    preferred_element_type=jnp.float32)
            + b2_ref[...]
        )
        h2_ref[pl.ds(i * _BM, _BM), :] = h2

    @pl.when(p == 1)
    def _():
        o_ref[...] = jnp.dot(a_ref[...], h2_ref[...],
                             preferred_element_type=jnp.float32)


def kernel(x, adj_t, W1, b1, W2, b2):
    w1t = W1.T
    w2t = W2.T
    b1r = b1.reshape(1, _D)
    b2r = b2.reshape(1, _D)

    return pl.pallas_call(
        _gcn_kernel,
        grid=(2, _N // _BM),
        out_shape=jax.ShapeDtypeStruct((_N, _D), jnp.float32),
        in_specs=[
            pl.BlockSpec((_N, _D), lambda p, i: (0, 0)),
            pl.BlockSpec((_BM, _N), lambda p, i: (i, 0)),
            pl.BlockSpec((_D, _D), lambda p, i: (0, 0)),
            pl.BlockSpec((1, _D), lambda p, i: (0, 0)),
            pl.BlockSpec((_D, _D), lambda p, i: (0, 0)),
            pl.BlockSpec((1, _D), lambda p, i: (0, 0)),
        ],
        out_specs=pl.BlockSpec((_BM, _D), lambda p, i: (p * i, 0)),
        scratch_shapes=[
            pltpu.VMEM((_N, _D), jnp.float32),
            pltpu.VMEM((_N, _D), jnp.float32),
        ],
        compiler_params=pltpu.CompilerParams(
            dimension_semantics=("arbitrary", "arbitrary"),
            vmem_limit_bytes=100 * 1024 * 1024,
        ),
    )(x, adj_t, w1t, b1r, w2t, b2r)
